# baseline (speedup 1.0000x reference)
import functools

import jax
import jax.numpy as jnp
from jax import lax
from jax.experimental import pallas as pl
from jax.experimental.pallas import tpu as pltpu
from jax.experimental.pallas import tpu_sc as plsc

N_ADDR = 1000000
D = 64
TEMPERATURE = 0.1
EPS = 1e-8

NW = 32
CHUNK = 400
GROUPS = CHUNK // 16
NCHUNK = N_ADDR // CHUNK
BASE_TRIPS = NCHUNK // NW
EXTRA_W = NCHUNK - BASE_TRIPS * NW


def _vgather(x, idx):
    return lax.gather(
        x, idx[:, None],
        lax.GatherDimensionNumbers(
            offset_dims=(), collapsed_slice_dims=(0,), start_index_map=(0,)),
        slice_sizes=(1,),
        mode=lax.GatherScatterMode.PROMISE_IN_BOUNDS)


def _hsum(x):
    lanes = lax.iota(jnp.int32, 16)
    sixteen = jnp.full((16,), 16, jnp.int32)
    for sh in (8, 4, 2, 1):
        x = x + _vgather(x, lax.rem(lanes + sh, sixteen))
    return x


def _rsqrt(x):
    xi = lax.bitcast_convert_type(x, jnp.int32)
    yi = jnp.int32(0x5F3759DF) - lax.shift_right_arithmetic(xi, 1)
    y = lax.bitcast_convert_type(yi, jnp.float32)
    for _ in range(3):
        y = y * (1.5 - 0.5 * x * y * y)
    return y


def _sc_body(q_hbm, a_hbm, outv_hbm, outs_hbm, qbuf, buf0, buf1, vbuf, sbuf,
             sem0, sem1):
    wid = lax.axis_index("s") * 2 + lax.axis_index("c")

    pltpu.sync_copy(q_hbm.at[0], qbuf)
    q0 = qbuf[pl.ds(0, 16)]
    q1 = qbuf[pl.ds(16, 16)]
    q2 = qbuf[pl.ds(32, 16)]
    q3 = qbuf[pl.ds(48, 16)]
    qss = _hsum(q0 * q0 + q1 * q1 + q2 * q2 + q3 * q3)

    lanes = lax.iota(jnp.int32, 16)

    def chunk_rows(t):
        return (wid + NW * t) * CHUNK

    def _start(buf, sem, t):
        pltpu.make_async_copy(
            a_hbm.at[pl.ds(chunk_rows(t), CHUNK), :], buf, sem).start()

    def _wait(buf, sem, t):
        pltpu.make_async_copy(
            a_hbm.at[pl.ds(chunk_rows(t), CHUNK), :], buf, sem).wait()

    def make_group_body(buf):
        def group_body(g, carry):
            va0, va1, va2, va3, sacc = carry
            base = g * 16
            dacc = jnp.zeros((16,), jnp.float32)
            ssacc = jnp.zeros((16,), jnp.float32)
            for j in range(16):
                a0 = buf[base + j, pl.ds(0, 16)]
                a1 = buf[base + j, pl.ds(16, 16)]
                a2 = buf[base + j, pl.ds(32, 16)]
                a3 = buf[base + j, pl.ds(48, 16)]
                dh = _hsum(a0 * q0 + a1 * q1 + a2 * q2 + a3 * q3)
                sh = _hsum(a0 * a0 + a1 * a1 + a2 * a2 + a3 * a3)
                sel = lanes == j
                dacc = jnp.where(sel, dh, dacc)
                ssacc = jnp.where(sel, sh, ssacc)
            x = ssacc * qss
            nrm = x * _rsqrt(x)
            sim = dacc / jnp.maximum(nrm, EPS)
            w = jnp.exp((sim - 1.0) * (1.0 / TEMPERATURE))
            for j in range(16):
                wj = _vgather(w, jnp.full((16,), j, jnp.int32))
                a0 = buf[base + j, pl.ds(0, 16)]
                a1 = buf[base + j, pl.ds(16, 16)]
                a2 = buf[base + j, pl.ds(32, 16)]
                a3 = buf[base + j, pl.ds(48, 16)]
                va0 = va0 + wj * a0
                va1 = va1 + wj * a1
                va2 = va2 + wj * a2
                va3 = va3 + wj * a3
            return (va0, va1, va2, va3, sacc + w)
        return group_body

    gb0 = make_group_body(buf0)
    gb1 = make_group_body(buf1)

    def pair_body(p, carry):
        t0 = 2 * p
        pltpu.sync_copy(a_hbm.at[pl.ds(chunk_rows(t0), CHUNK), :], buf0)
        carry = lax.fori_loop(0, GROUPS, gb0, carry)
        pltpu.sync_copy(a_hbm.at[pl.ds(chunk_rows(t0 + 1), CHUNK), :], buf1)
        return lax.fori_loop(0, GROUPS, gb1, carry)

    z = jnp.zeros((16,), jnp.float32)
    carry = lax.fori_loop(0, BASE_TRIPS // 2, pair_body, (z, z, z, z, z))

    def extra_body(_, carry):
        pltpu.sync_copy(
            a_hbm.at[pl.ds((NW * BASE_TRIPS + wid) * CHUNK, CHUNK), :], buf0)
        return lax.fori_loop(0, GROUPS, gb0, carry)

    n_extra = jnp.where(wid < EXTRA_W, 1, 0)
    va0, va1, va2, va3, sacc = lax.fori_loop(0, n_extra, extra_body, carry)

    vbuf[pl.ds(0, 16)] = va0
    vbuf[pl.ds(16, 16)] = va1
    vbuf[pl.ds(32, 16)] = va2
    vbuf[pl.ds(48, 16)] = va3
    sbuf[...] = sacc
    pltpu.sync_copy(vbuf, outv_hbm.at[wid])
    pltpu.sync_copy(sbuf, outs_hbm.at[wid])


@jax.jit
def kernel(query_address, addresses):
    mesh = plsc.VectorSubcoreMesh(core_axis_name="c", subcore_axis_name="s")
    run = functools.partial(
        pl.kernel,
        mesh=mesh,
        out_type=[
            jax.ShapeDtypeStruct((NW, D), jnp.float32),
            jax.ShapeDtypeStruct((NW, 16), jnp.float32),
        ],
        scratch_types=[
            pltpu.VMEM((D,), jnp.float32),
            pltpu.VMEM((CHUNK, D), jnp.float32),
            pltpu.VMEM((CHUNK, D), jnp.float32),
            pltpu.VMEM((D,), jnp.float32),
            pltpu.VMEM((16,), jnp.float32),
            pltpu.SemaphoreType.DMA,
            pltpu.SemaphoreType.DMA,
        ],
    )(_sc_body)
    outv, outs = run(query_address, addresses)
    wsum = jnp.sum(outv, axis=0)
    ssum = jnp.sum(outs)
    return wsum / ssum


# speedup vs baseline: 1.5320x; 1.5320x over previous
import functools

import jax
import jax.numpy as jnp
from jax import lax
from jax.experimental import pallas as pl
from jax.experimental.pallas import tpu as pltpu
from jax.experimental.pallas import tpu_sc as plsc

N_ADDR = 1000000
D = 64
TEMPERATURE = 0.1
EPS = 1e-8

SC_START = 500000

NW = 32
CHUNK = 400
SC_C0 = SC_START // CHUNK
SC_NCHUNK = (N_ADDR - SC_START) // CHUNK
SC_TRIPS = SC_NCHUNK // NW
SC_EXTRA = SC_NCHUNK - SC_TRIPS * NW
GROUPS = CHUNK // 16

TC_CHUNK = 10000
TC_NCHUNK = SC_START // TC_CHUNK
NBUF = 8


def _vgather(x, idx):
    return lax.gather(
        x, idx[:, None],
        lax.GatherDimensionNumbers(
            offset_dims=(), collapsed_slice_dims=(0,), start_index_map=(0,)),
        slice_sizes=(1,),
        mode=lax.GatherScatterMode.PROMISE_IN_BOUNDS)


def _hsum(x):
    lanes = lax.iota(jnp.int32, 16)
    sixteen = jnp.full((16,), 16, jnp.int32)
    for sh in (8, 4, 2, 1):
        x = x + _vgather(x, lax.rem(lanes + sh, sixteen))
    return x


def _rsqrt(x):
    xi = lax.bitcast_convert_type(x, jnp.int32)
    yi = jnp.int32(0x5F3759DF) - lax.shift_right_arithmetic(xi, 1)
    y = lax.bitcast_convert_type(yi, jnp.float32)
    for _ in range(3):
        y = y * (1.5 - 0.5 * x * y * y)
    return y


def _sc_body(q_hbm, a_hbm, outv_hbm, outs_hbm, qbuf, buf0, vbuf, sbuf):
    wid = lax.axis_index("s") * 2 + lax.axis_index("c")

    pltpu.sync_copy(q_hbm.at[0], qbuf)
    q0 = qbuf[pl.ds(0, 16)]
    q1 = qbuf[pl.ds(16, 16)]
    q2 = qbuf[pl.ds(32, 16)]
    q3 = qbuf[pl.ds(48, 16)]
    qss = _hsum(q0 * q0 + q1 * q1 + q2 * q2 + q3 * q3)

    lanes = lax.iota(jnp.int32, 16)

    def group_body(g, carry):
        va0, va1, va2, va3, sacc = carry
        base = g * 16
        dacc = jnp.zeros((16,), jnp.float32)
        ssacc = jnp.zeros((16,), jnp.float32)
        for j in range(16):
            a0 = buf0[base + j, pl.ds(0, 16)]
            a1 = buf0[base + j, pl.ds(16, 16)]
            a2 = buf0[base + j, pl.ds(32, 16)]
            a3 = buf0[base + j, pl.ds(48, 16)]
            dh = _hsum(a0 * q0 + a1 * q1 + a2 * q2 + a3 * q3)
            sh = _hsum(a0 * a0 + a1 * a1 + a2 * a2 + a3 * a3)
            sel = lanes == j
            dacc = jnp.where(sel, dh, dacc)
            ssacc = jnp.where(sel, sh, ssacc)
        x = ssacc * qss
        nrm = x * _rsqrt(x)
        sim = dacc / jnp.maximum(nrm, EPS)
        w = jnp.exp((sim - 1.0) * (1.0 / TEMPERATURE))
        for j in range(16):
            wj = _vgather(w, jnp.full((16,), j, jnp.int32))
            a0 = buf0[base + j, pl.ds(0, 16)]
            a1 = buf0[base + j, pl.ds(16, 16)]
            a2 = buf0[base + j, pl.ds(32, 16)]
            a3 = buf0[base + j, pl.ds(48, 16)]
            va0 = va0 + wj * a0
            va1 = va1 + wj * a1
            va2 = va2 + wj * a2
            va3 = va3 + wj * a3
        return (va0, va1, va2, va3, sacc + w)

    def chunk_body(t, carry):
        c = SC_C0 + wid + NW * t
        pltpu.sync_copy(a_hbm.at[pl.ds(c * CHUNK, CHUNK), :], buf0)
        return lax.fori_loop(0, GROUPS, group_body, carry)

    z = jnp.zeros((16,), jnp.float32)
    carry = lax.fori_loop(0, SC_TRIPS, chunk_body, (z, z, z, z, z))

    def extra_body(_, carry):
        c = SC_C0 + NW * SC_TRIPS + wid
        pltpu.sync_copy(a_hbm.at[pl.ds(c * CHUNK, CHUNK), :], buf0)
        return lax.fori_loop(0, GROUPS, group_body, carry)

    n_extra = jnp.where(wid < SC_EXTRA, 1, 0)
    va0, va1, va2, va3, sacc = lax.fori_loop(0, n_extra, extra_body, carry)

    vbuf[pl.ds(0, 16)] = va0
    vbuf[pl.ds(16, 16)] = va1
    vbuf[pl.ds(32, 16)] = va2
    vbuf[pl.ds(48, 16)] = va3
    sbuf[...] = sacc
    pltpu.sync_copy(vbuf, outv_hbm.at[wid])
    pltpu.sync_copy(sbuf, outs_hbm.at[wid])


def _tc_copy(a_hbm, buf, sem, c):
    return pltpu.make_async_copy(
        a_hbm.at[pl.ds(c * TC_CHUNK, TC_CHUNK), :], buf, sem)


def _tc_body(q_ref, a_hbm, wsum_ref, ssum_ref, bufs, sems):
    i = pl.program_id(0)

    @pl.when(i == 0)
    def _prime():
        wsum_ref[...] = jnp.zeros_like(wsum_ref)
        ssum_ref[...] = jnp.zeros_like(ssum_ref)
        for k in range(NBUF - 1):
            _tc_copy(a_hbm, bufs.at[k], sems.at[k], k).start()

    nxt = i + NBUF - 1

    @pl.when(nxt < TC_NCHUNK)
    def _ahead():
        _tc_copy(a_hbm, bufs.at[nxt % NBUF], sems.at[nxt % NBUF], nxt).start()

    _tc_copy(a_hbm, bufs.at[i % NBUF], sems.at[i % NBUF], i).wait()
    a = bufs[i % NBUF]
    q = q_ref[...]

    dots = jax.lax.dot_general(
        q, a, (((1,), (1,)), ((), ())),
        preferred_element_type=jnp.float32)
    ones = jnp.ones((1, D), jnp.float32)
    sumsq = jax.lax.dot_general(
        ones, a * a, (((1,), (1,)), ((), ())),
        preferred_element_type=jnp.float32)

    qn = jnp.sqrt(jnp.sum(q * q))
    an = jnp.sqrt(sumsq)
    sim = dots / jnp.maximum(an * qn, EPS)
    w = jnp.exp((sim - 1.0) / TEMPERATURE)

    part = jax.lax.dot_general(
        w, a, (((1,), (0,)), ((), ())),
        preferred_element_type=jnp.float32)
    wsum_ref[...] += part
    ssum_ref[...] += jnp.sum(w)


@jax.jit
def kernel(query_address, addresses):
    mesh = plsc.VectorSubcoreMesh(core_axis_name="c", subcore_axis_name="s")
    sc_run = functools.partial(
        pl.kernel,
        mesh=mesh,
        out_type=[
            jax.ShapeDtypeStruct((NW, D), jnp.float32),
            jax.ShapeDtypeStruct((NW, 16), jnp.float32),
        ],
        scratch_types=[
            pltpu.VMEM((D,), jnp.float32),
            pltpu.VMEM((CHUNK, D), jnp.float32),
            pltpu.VMEM((D,), jnp.float32),
            pltpu.VMEM((16,), jnp.float32),
        ],
    )(_sc_body)
    outv, outs = sc_run(query_address, addresses)

    wsum_tc, ssum_tc = pl.pallas_call(
        _tc_body,
        grid=(TC_NCHUNK,),
        in_specs=[
            pl.BlockSpec((1, D), lambda i: (0, 0)),
            pl.BlockSpec(memory_space=pl.ANY),
        ],
        out_specs=[
            pl.BlockSpec((1, D), lambda i: (0, 0)),
            pl.BlockSpec((1, 1), lambda i: (0, 0)),
        ],
        out_shape=[
            jax.ShapeDtypeStruct((1, D), jnp.float32),
            jax.ShapeDtypeStruct((1, 1), jnp.float32),
        ],
        scratch_shapes=[
            pltpu.VMEM((NBUF, TC_CHUNK, D), jnp.float32),
            pltpu.SemaphoreType.DMA((NBUF,)),
        ],
    )(query_address, addresses)

    wsum = jnp.sum(outv, axis=0) + wsum_tc[0]
    ssum = jnp.sum(outs) + ssum_tc[0, 0]
    return wsum / ssum


# speedup vs baseline: 2.2085x; 1.4416x over previous
import functools

import jax
import jax.numpy as jnp
from jax import lax
from jax.experimental import pallas as pl
from jax.experimental.pallas import tpu as pltpu
from jax.experimental.pallas import tpu_sc as plsc

N_ADDR = 1000000
D = 64
TEMPERATURE = 0.1
EPS = 1e-8

SC_START = 750000

NW = 32
CHUNK = 400
SC_C0 = SC_START // CHUNK
SC_NCHUNK = (N_ADDR - SC_START) // CHUNK
SC_TRIPS = SC_NCHUNK // NW
SC_EXTRA = SC_NCHUNK - SC_TRIPS * NW
GROUPS = CHUNK // 16

TC_CHUNK = 10000
TC_NCHUNK = SC_START // TC_CHUNK
NBUF = 8


def _vgather(x, idx):
    return lax.gather(
        x, idx[:, None],
        lax.GatherDimensionNumbers(
            offset_dims=(), collapsed_slice_dims=(0,), start_index_map=(0,)),
        slice_sizes=(1,),
        mode=lax.GatherScatterMode.PROMISE_IN_BOUNDS)


def _hsum(x):
    lanes = lax.iota(jnp.int32, 16)
    sixteen = jnp.full((16,), 16, jnp.int32)
    for sh in (8, 4, 2, 1):
        x = x + _vgather(x, lax.rem(lanes + sh, sixteen))
    return x


def _rsqrt(x):
    xi = lax.bitcast_convert_type(x, jnp.int32)
    yi = jnp.int32(0x5F3759DF) - lax.shift_right_arithmetic(xi, 1)
    y = lax.bitcast_convert_type(yi, jnp.float32)
    for _ in range(3):
        y = y * (1.5 - 0.5 * x * y * y)
    return y


def _sc_body(q_hbm, a_hbm, outv_hbm, outs_hbm, qbuf, buf0, vbuf, sbuf):
    wid = lax.axis_index("s") * 2 + lax.axis_index("c")

    pltpu.sync_copy(q_hbm.at[0], qbuf)
    q0 = qbuf[pl.ds(0, 16)]
    q1 = qbuf[pl.ds(16, 16)]
    q2 = qbuf[pl.ds(32, 16)]
    q3 = qbuf[pl.ds(48, 16)]
    qss = _hsum(q0 * q0 + q1 * q1 + q2 * q2 + q3 * q3)

    lanes = lax.iota(jnp.int32, 16)

    def group_body(g, carry):
        va0, va1, va2, va3, sacc = carry
        base = g * 16
        dacc = jnp.zeros((16,), jnp.float32)
        ssacc = jnp.zeros((16,), jnp.float32)
        for j in range(16):
            a0 = buf0[base + j, pl.ds(0, 16)]
            a1 = buf0[base + j, pl.ds(16, 16)]
            a2 = buf0[base + j, pl.ds(32, 16)]
            a3 = buf0[base + j, pl.ds(48, 16)]
            dh = _hsum(a0 * q0 + a1 * q1 + a2 * q2 + a3 * q3)
            sh = _hsum(a0 * a0 + a1 * a1 + a2 * a2 + a3 * a3)
            sel = lanes == j
            dacc = jnp.where(sel, dh, dacc)
            ssacc = jnp.where(sel, sh, ssacc)
        x = ssacc * qss
        nrm = x * _rsqrt(x)
        sim = dacc / jnp.maximum(nrm, EPS)
        w = jnp.exp((sim - 1.0) * (1.0 / TEMPERATURE))
        for j in range(16):
            wj = _vgather(w, jnp.full((16,), j, jnp.int32))
            a0 = buf0[base + j, pl.ds(0, 16)]
            a1 = buf0[base + j, pl.ds(16, 16)]
            a2 = buf0[base + j, pl.ds(32, 16)]
            a3 = buf0[base + j, pl.ds(48, 16)]
            va0 = va0 + wj * a0
            va1 = va1 + wj * a1
            va2 = va2 + wj * a2
            va3 = va3 + wj * a3
        return (va0, va1, va2, va3, sacc + w)

    def chunk_body(t, carry):
        c = SC_C0 + wid + NW * t
        pltpu.sync_copy(a_hbm.at[pl.ds(c * CHUNK, CHUNK), :], buf0)
        return lax.fori_loop(0, GROUPS, group_body, carry)

    z = jnp.zeros((16,), jnp.float32)
    carry = lax.fori_loop(0, SC_TRIPS, chunk_body, (z, z, z, z, z))

    def extra_body(_, carry):
        c = SC_C0 + NW * SC_TRIPS + wid
        pltpu.sync_copy(a_hbm.at[pl.ds(c * CHUNK, CHUNK), :], buf0)
        return lax.fori_loop(0, GROUPS, group_body, carry)

    n_extra = jnp.where(wid < SC_EXTRA, 1, 0)
    va0, va1, va2, va3, sacc = lax.fori_loop(0, n_extra, extra_body, carry)

    vbuf[pl.ds(0, 16)] = va0
    vbuf[pl.ds(16, 16)] = va1
    vbuf[pl.ds(32, 16)] = va2
    vbuf[pl.ds(48, 16)] = va3
    sbuf[...] = sacc
    pltpu.sync_copy(vbuf, outv_hbm.at[wid])
    pltpu.sync_copy(sbuf, outs_hbm.at[wid])


def _tc_copy(a_hbm, buf, sem, c):
    return pltpu.make_async_copy(
        a_hbm.at[pl.ds(c * TC_CHUNK, TC_CHUNK), :], buf, sem)


def _tc_body(q_ref, a_hbm, wsum_ref, ssum_ref, bufs, sems):
    i = pl.program_id(0)

    @pl.when(i == 0)
    def _prime():
        wsum_ref[...] = jnp.zeros_like(wsum_ref)
        ssum_ref[...] = jnp.zeros_like(ssum_ref)
        for k in range(NBUF - 1):
            _tc_copy(a_hbm, bufs.at[k], sems.at[k], k).start()

    nxt = i + NBUF - 1

    @pl.when(nxt < TC_NCHUNK)
    def _ahead():
        _tc_copy(a_hbm, bufs.at[nxt % NBUF], sems.at[nxt % NBUF], nxt).start()

    _tc_copy(a_hbm, bufs.at[i % NBUF], sems.at[i % NBUF], i).wait()
    a = bufs[i % NBUF]
    q = q_ref[...]

    dots = jax.lax.dot_general(
        q, a, (((1,), (1,)), ((), ())),
        preferred_element_type=jnp.float32)
    ones = jnp.ones((1, D), jnp.float32)
    sumsq = jax.lax.dot_general(
        ones, a * a, (((1,), (1,)), ((), ())),
        preferred_element_type=jnp.float32)

    qn = jnp.sqrt(jnp.sum(q * q))
    an = jnp.sqrt(sumsq)
    sim = dots / jnp.maximum(an * qn, EPS)
    w = jnp.exp((sim - 1.0) / TEMPERATURE)

    part = jax.lax.dot_general(
        w, a, (((1,), (0,)), ((), ())),
        preferred_element_type=jnp.float32)
    wsum_ref[...] += part
    ssum_ref[...] += jnp.sum(w)


@jax.jit
def kernel(query_address, addresses):
    mesh = plsc.VectorSubcoreMesh(core_axis_name="c", subcore_axis_name="s")
    sc_run = functools.partial(
        pl.kernel,
        mesh=mesh,
        out_type=[
            jax.ShapeDtypeStruct((NW, D), jnp.float32),
            jax.ShapeDtypeStruct((NW, 16), jnp.float32),
        ],
        scratch_types=[
            pltpu.VMEM((D,), jnp.float32),
            pltpu.VMEM((CHUNK, D), jnp.float32),
            pltpu.VMEM((D,), jnp.float32),
            pltpu.VMEM((16,), jnp.float32),
        ],
    )(_sc_body)
    outv, outs = sc_run(query_address, addresses)

    wsum_tc, ssum_tc = pl.pallas_call(
        _tc_body,
        grid=(TC_NCHUNK,),
        in_specs=[
            pl.BlockSpec((1, D), lambda i: (0, 0)),
            pl.BlockSpec(memory_space=pl.ANY),
        ],
        out_specs=[
            pl.BlockSpec((1, D), lambda i: (0, 0)),
            pl.BlockSpec((1, 1), lambda i: (0, 0)),
        ],
        out_shape=[
            jax.ShapeDtypeStruct((1, D), jnp.float32),
            jax.ShapeDtypeStruct((1, 1), jnp.float32),
        ],
        scratch_shapes=[
            pltpu.VMEM((NBUF, TC_CHUNK, D), jnp.float32),
            pltpu.SemaphoreType.DMA((NBUF,)),
        ],
    )(query_address, addresses)

    wsum = jnp.sum(outv, axis=0) + wsum_tc[0]
    ssum = jnp.sum(outs) + ssum_tc[0, 0]
    return wsum / ssum


# speedup vs baseline: 2.4522x; 1.1104x over previous
import functools

import jax
import jax.numpy as jnp
from jax import lax
from jax.experimental import pallas as pl
from jax.experimental.pallas import tpu as pltpu
from jax.experimental.pallas import tpu_sc as plsc

N_ADDR = 1000000
D = 64
TEMPERATURE = 0.1
EPS = 1e-8

SC_START = 744000

NW = 32
CHUNK = 400
SC_C0 = SC_START // CHUNK
SC_NCHUNK = (N_ADDR - SC_START) // CHUNK
SC_TRIPS = SC_NCHUNK // NW
GROUPS = CHUNK // 16

TC_CHUNK = 8000
TC_NCHUNK = SC_START // TC_CHUNK
NBUF = 8


def _vgather(x, idx):
    return lax.gather(
        x, idx[:, None],
        lax.GatherDimensionNumbers(
            offset_dims=(), collapsed_slice_dims=(0,), start_index_map=(0,)),
        slice_sizes=(1,),
        mode=lax.GatherScatterMode.PROMISE_IN_BOUNDS)


def _hsum(x):
    lanes = lax.iota(jnp.int32, 16)
    sixteen = jnp.full((16,), 16, jnp.int32)
    for sh in (8, 4, 2, 1):
        x = x + _vgather(x, lax.rem(lanes + sh, sixteen))
    return x


def _rsqrt(x):
    xi = lax.bitcast_convert_type(x, jnp.int32)
    yi = jnp.int32(0x5F3759DF) - lax.shift_right_arithmetic(xi, 1)
    y = lax.bitcast_convert_type(yi, jnp.float32)
    for _ in range(3):
        y = y * (1.5 - 0.5 * x * y * y)
    return y


def _sc_body(q_hbm, a_hbm, outv_hbm, outs_hbm, qbuf, buf0, buf1, vbuf, sbuf,
             sem0, sem1):
    wid = lax.axis_index("s") * 2 + lax.axis_index("c")

    pltpu.sync_copy(q_hbm.at[0], qbuf)
    q0 = qbuf[pl.ds(0, 16)]
    q1 = qbuf[pl.ds(16, 16)]
    q2 = qbuf[pl.ds(32, 16)]
    q3 = qbuf[pl.ds(48, 16)]
    qss = _hsum(q0 * q0 + q1 * q1 + q2 * q2 + q3 * q3)

    lanes = lax.iota(jnp.int32, 16)

    def chunk_rows(t):
        return (SC_C0 + wid + NW * t) * CHUNK

    def _start(buf, sem, t):
        pltpu.make_async_copy(
            a_hbm.at[pl.ds(chunk_rows(t), CHUNK), :], buf, sem).start()

    def _wait(buf, sem, t):
        pltpu.make_async_copy(
            a_hbm.at[pl.ds(chunk_rows(t), CHUNK), :], buf, sem).wait()

    def make_group_body(buf0):
        def group_body(g, carry):
            va0, va1, va2, va3, sacc = carry
            base = g * 16
            dacc = jnp.zeros((16,), jnp.float32)
            ssacc = jnp.zeros((16,), jnp.float32)
            for j in range(16):
                a0 = buf0[base + j, pl.ds(0, 16)]
                a1 = buf0[base + j, pl.ds(16, 16)]
                a2 = buf0[base + j, pl.ds(32, 16)]
                a3 = buf0[base + j, pl.ds(48, 16)]
                dh = _hsum(a0 * q0 + a1 * q1 + a2 * q2 + a3 * q3)
                sh = _hsum(a0 * a0 + a1 * a1 + a2 * a2 + a3 * a3)
                sel = lanes == j
                dacc = jnp.where(sel, dh, dacc)
                ssacc = jnp.where(sel, sh, ssacc)
            x = ssacc * qss
            nrm = x * _rsqrt(x)
            sim = dacc / jnp.maximum(nrm, EPS)
            w = jnp.exp((sim - 1.0) * (1.0 / TEMPERATURE))
            for j in range(16):
                wj = _vgather(w, jnp.full((16,), j, jnp.int32))
                a0 = buf0[base + j, pl.ds(0, 16)]
                a1 = buf0[base + j, pl.ds(16, 16)]
                a2 = buf0[base + j, pl.ds(32, 16)]
                a3 = buf0[base + j, pl.ds(48, 16)]
                va0 = va0 + wj * a0
                va1 = va1 + wj * a1
                va2 = va2 + wj * a2
                va3 = va3 + wj * a3
            return (va0, va1, va2, va3, sacc + w)
        return group_body

    gb0 = make_group_body(buf0)
    gb1 = make_group_body(buf1)

    _start(buf0, sem0, 0)

    def pair_body(p, carry):
        t0 = 2 * p
        _start(buf1, sem1, t0 + 1)
        _wait(buf0, sem0, t0)
        carry = lax.fori_loop(0, GROUPS, gb0, carry)

        @pl.when(t0 + 2 < SC_TRIPS)
        def _():
            _start(buf0, sem0, t0 + 2)

        _wait(buf1, sem1, t0 + 1)
        return lax.fori_loop(0, GROUPS, gb1, carry)

    z = jnp.zeros((16,), jnp.float32)
    va0, va1, va2, va3, sacc = lax.fori_loop(
        0, SC_TRIPS // 2, pair_body, (z, z, z, z, z))

    vbuf[pl.ds(0, 16)] = va0
    vbuf[pl.ds(16, 16)] = va1
    vbuf[pl.ds(32, 16)] = va2
    vbuf[pl.ds(48, 16)] = va3
    sbuf[...] = sacc
    pltpu.sync_copy(vbuf, outv_hbm.at[wid])
    pltpu.sync_copy(sbuf, outs_hbm.at[wid])


def _tc_copy(a_hbm, buf, sem, c):
    return pltpu.make_async_copy(
        a_hbm.at[pl.ds(c * TC_CHUNK, TC_CHUNK), :], buf, sem)


def _tc_body(q_ref, a_hbm, wsum_ref, ssum_ref, bufs, sems):
    i = pl.program_id(0)

    @pl.when(i == 0)
    def _prime():
        wsum_ref[...] = jnp.zeros_like(wsum_ref)
        ssum_ref[...] = jnp.zeros_like(ssum_ref)
        for k in range(NBUF - 1):
            _tc_copy(a_hbm, bufs.at[k], sems.at[k], k).start()

    nxt = i + NBUF - 1

    @pl.when(nxt < TC_NCHUNK)
    def _ahead():
        _tc_copy(a_hbm, bufs.at[nxt % NBUF], sems.at[nxt % NBUF], nxt).start()

    _tc_copy(a_hbm, bufs.at[i % NBUF], sems.at[i % NBUF], i).wait()
    a = bufs[i % NBUF]
    q = q_ref[...]

    dots = jax.lax.dot_general(
        q, a, (((1,), (1,)), ((), ())),
        preferred_element_type=jnp.float32)
    ones = jnp.ones((1, D), jnp.float32)
    sumsq = jax.lax.dot_general(
        ones, a * a, (((1,), (1,)), ((), ())),
        preferred_element_type=jnp.float32)

    qn = jnp.sqrt(jnp.sum(q * q))
    an = jnp.sqrt(sumsq)
    sim = dots / jnp.maximum(an * qn, EPS)
    w = jnp.exp((sim - 1.0) / TEMPERATURE)

    part = jax.lax.dot_general(
        w, a, (((1,), (0,)), ((), ())),
        preferred_element_type=jnp.float32)
    wsum_ref[...] += part
    ssum_ref[...] += jnp.sum(w)


@jax.jit
def kernel(query_address, addresses):
    mesh = plsc.VectorSubcoreMesh(core_axis_name="c", subcore_axis_name="s")
    sc_run = functools.partial(
        pl.kernel,
        mesh=mesh,
        out_type=[
            jax.ShapeDtypeStruct((NW, D), jnp.float32),
            jax.ShapeDtypeStruct((NW, 16), jnp.float32),
        ],
        scratch_types=[
            pltpu.VMEM((D,), jnp.float32),
            pltpu.VMEM((CHUNK, D), jnp.float32),
            pltpu.VMEM((CHUNK, D), jnp.float32),
            pltpu.VMEM((D,), jnp.float32),
            pltpu.VMEM((16,), jnp.float32),
            pltpu.SemaphoreType.DMA,
            pltpu.SemaphoreType.DMA,
        ],
    )(_sc_body)
    outv, outs = sc_run(query_address, addresses)

    wsum_tc, ssum_tc = pl.pallas_call(
        _tc_body,
        grid=(TC_NCHUNK,),
        in_specs=[
            pl.BlockSpec((1, D), lambda i: (0, 0)),
            pl.BlockSpec(memory_space=pl.ANY),
        ],
        out_specs=[
            pl.BlockSpec((1, D), lambda i: (0, 0)),
            pl.BlockSpec((1, 1), lambda i: (0, 0)),
        ],
        out_shape=[
            jax.ShapeDtypeStruct((1, D), jnp.float32),
            jax.ShapeDtypeStruct((1, 1), jnp.float32),
        ],
        scratch_shapes=[
            pltpu.VMEM((NBUF, TC_CHUNK, D), jnp.float32),
            pltpu.SemaphoreType.DMA((NBUF,)),
        ],
    )(query_address, addresses)

    wsum = jnp.sum(outv, axis=0) + wsum_tc[0]
    ssum = jnp.sum(outs) + ssum_tc[0, 0]
    return wsum / ssum


# speedup vs baseline: 2.8526x; 1.1633x over previous
import functools

import jax
import jax.numpy as jnp
from jax import lax
from jax.experimental import pallas as pl
from jax.experimental.pallas import tpu as pltpu
from jax.experimental.pallas import tpu_sc as plsc

N_ADDR = 1000000
D = 64
TEMPERATURE = 0.1
EPS = 1e-8

SC_START = 744000

NW = 32
CHUNK = 400
SC_C0 = SC_START // CHUNK
SC_NCHUNK = (N_ADDR - SC_START) // CHUNK
SC_TRIPS = SC_NCHUNK // NW
GROUPS = CHUNK // 8

TC_CHUNK = 8000
TC_NCHUNK = SC_START // TC_CHUNK
NBUF = 8


def _vgather(x, idx):
    return lax.gather(
        x, idx[:, None],
        lax.GatherDimensionNumbers(
            offset_dims=(), collapsed_slice_dims=(0,), start_index_map=(0,)),
        slice_sizes=(1,),
        mode=lax.GatherScatterMode.PROMISE_IN_BOUNDS)


def _hsum(x):
    lanes = lax.iota(jnp.int32, 16)
    sixteen = jnp.full((16,), 16, jnp.int32)
    for sh in (8, 4, 2, 1):
        x = x + _vgather(x, lax.rem(lanes + sh, sixteen))
    return x


def _rsqrt(x):
    xi = lax.bitcast_convert_type(x, jnp.int32)
    yi = jnp.int32(0x5F3759DF) - lax.shift_right_arithmetic(xi, 1)
    y = lax.bitcast_convert_type(yi, jnp.float32)
    for _ in range(3):
        y = y * (1.5 - 0.5 * x * y * y)
    return y


def _sc_body(q_hbm, a_hbm, outv_hbm, outs_hbm, qbuf, buf0, buf1, vbuf, sbuf,
             sem0, sem1):
    wid = lax.axis_index("s") * 2 + lax.axis_index("c")

    pltpu.sync_copy(q_hbm.at[0], qbuf)
    q0 = qbuf[pl.ds(0, 16)]
    q1 = qbuf[pl.ds(16, 16)]
    q2 = qbuf[pl.ds(32, 16)]
    q3 = qbuf[pl.ds(48, 16)]
    qss = _hsum(q0 * q0 + q1 * q1 + q2 * q2 + q3 * q3)

    lanes = lax.iota(jnp.int32, 16)

    def chunk_rows(t):
        return (SC_C0 + wid + NW * t) * CHUNK

    def _start(buf, sem, t):
        pltpu.make_async_copy(
            a_hbm.at[pl.ds(chunk_rows(t), CHUNK), :], buf, sem).start()

    def _wait(buf, sem, t):
        pltpu.make_async_copy(
            a_hbm.at[pl.ds(chunk_rows(t), CHUNK), :], buf, sem).wait()

    def make_group_body(buf0):
        def group_body(g, carry):
            va0, va1, va2, va3, sacc = carry
            base = g * 8
            dacc = jnp.zeros((16,), jnp.float32)
            ssacc = jnp.zeros((16,), jnp.float32)
            for j in range(8):
                a0 = buf0[base + j, pl.ds(0, 16)]
                a1 = buf0[base + j, pl.ds(16, 16)]
                a2 = buf0[base + j, pl.ds(32, 16)]
                a3 = buf0[base + j, pl.ds(48, 16)]
                dh = _hsum(a0 * q0 + a1 * q1 + a2 * q2 + a3 * q3)
                sh = _hsum(a0 * a0 + a1 * a1 + a2 * a2 + a3 * a3)
                sel = lanes == j
                dacc = jnp.where(sel, dh, dacc)
                ssacc = jnp.where(sel, sh, ssacc)
            x = ssacc * qss
            nrm = x * _rsqrt(x)
            sim = dacc / jnp.maximum(nrm, EPS)
            w = jnp.exp((sim - 1.0) * (1.0 / TEMPERATURE))
            for j in range(8):
                wj = _vgather(w, jnp.full((16,), j, jnp.int32))
                a0 = buf0[base + j, pl.ds(0, 16)]
                a1 = buf0[base + j, pl.ds(16, 16)]
                a2 = buf0[base + j, pl.ds(32, 16)]
                a3 = buf0[base + j, pl.ds(48, 16)]
                va0 = va0 + wj * a0
                va1 = va1 + wj * a1
                va2 = va2 + wj * a2
                va3 = va3 + wj * a3
            return (va0, va1, va2, va3,
                    sacc + jnp.where(lanes < 8, w, 0.0))
        return group_body

    gb0 = make_group_body(buf0)
    gb1 = make_group_body(buf1)

    _start(buf0, sem0, 0)

    def pair_body(p, carry):
        t0 = 2 * p
        _start(buf1, sem1, t0 + 1)
        _wait(buf0, sem0, t0)
        carry = lax.fori_loop(0, GROUPS, gb0, carry)

        @pl.when(t0 + 2 < SC_TRIPS)
        def _():
            _start(buf0, sem0, t0 + 2)

        _wait(buf1, sem1, t0 + 1)
        return lax.fori_loop(0, GROUPS, gb1, carry)

    z = jnp.zeros((16,), jnp.float32)
    va0, va1, va2, va3, sacc = lax.fori_loop(
        0, SC_TRIPS // 2, pair_body, (z, z, z, z, z))

    vbuf[pl.ds(0, 16)] = va0
    vbuf[pl.ds(16, 16)] = va1
    vbuf[pl.ds(32, 16)] = va2
    vbuf[pl.ds(48, 16)] = va3
    sbuf[...] = sacc
    pltpu.sync_copy(vbuf, outv_hbm.at[wid])
    pltpu.sync_copy(sbuf, outs_hbm.at[wid])


def _tc_copy(a_hbm, buf, sem, c):
    return pltpu.make_async_copy(
        a_hbm.at[pl.ds(c * TC_CHUNK, TC_CHUNK), :], buf, sem)


def _tc_body(q_ref, a_hbm, wsum_ref, ssum_ref, bufs, sems):
    i = pl.program_id(0)

    @pl.when(i == 0)
    def _prime():
        wsum_ref[...] = jnp.zeros_like(wsum_ref)
        ssum_ref[...] = jnp.zeros_like(ssum_ref)
        for k in range(NBUF - 1):
            _tc_copy(a_hbm, bufs.at[k], sems.at[k], k).start()

    nxt = i + NBUF - 1

    @pl.when(nxt < TC_NCHUNK)
    def _ahead():
        _tc_copy(a_hbm, bufs.at[nxt % NBUF], sems.at[nxt % NBUF], nxt).start()

    _tc_copy(a_hbm, bufs.at[i % NBUF], sems.at[i % NBUF], i).wait()
    a = bufs[i % NBUF]
    q = q_ref[...]

    dots = jax.lax.dot_general(
        q, a, (((1,), (1,)), ((), ())),
        preferred_element_type=jnp.float32)
    ones = jnp.ones((1, D), jnp.float32)
    sumsq = jax.lax.dot_general(
        ones, a * a, (((1,), (1,)), ((), ())),
        preferred_element_type=jnp.float32)

    qn = jnp.sqrt(jnp.sum(q * q))
    an = jnp.sqrt(sumsq)
    sim = dots / jnp.maximum(an * qn, EPS)
    w = jnp.exp((sim - 1.0) / TEMPERATURE)

    part = jax.lax.dot_general(
        w, a, (((1,), (0,)), ((), ())),
        preferred_element_type=jnp.float32)
    wsum_ref[...] += part
    ssum_ref[...] += jnp.sum(w)


@jax.jit
def kernel(query_address, addresses):
    mesh = plsc.VectorSubcoreMesh(core_axis_name="c", subcore_axis_name="s")
    sc_run = functools.partial(
        pl.kernel,
        mesh=mesh,
        out_type=[
            jax.ShapeDtypeStruct((NW, D), jnp.float32),
            jax.ShapeDtypeStruct((NW, 16), jnp.float32),
        ],
        scratch_types=[
            pltpu.VMEM((D,), jnp.float32),
            pltpu.VMEM((CHUNK, D), jnp.float32),
            pltpu.VMEM((CHUNK, D), jnp.float32),
            pltpu.VMEM((D,), jnp.float32),
            pltpu.VMEM((16,), jnp.float32),
            pltpu.SemaphoreType.DMA,
            pltpu.SemaphoreType.DMA,
        ],
    )(_sc_body)
    outv, outs = sc_run(query_address, addresses)

    wsum_tc, ssum_tc = pl.pallas_call(
        _tc_body,
        grid=(TC_NCHUNK,),
        in_specs=[
            pl.BlockSpec((1, D), lambda i: (0, 0)),
            pl.BlockSpec(memory_space=pl.ANY),
        ],
        out_specs=[
            pl.BlockSpec((1, D), lambda i: (0, 0)),
            pl.BlockSpec((1, 1), lambda i: (0, 0)),
        ],
        out_shape=[
            jax.ShapeDtypeStruct((1, D), jnp.float32),
            jax.ShapeDtypeStruct((1, 1), jnp.float32),
        ],
        scratch_shapes=[
            pltpu.VMEM((NBUF, TC_CHUNK, D), jnp.float32),
            pltpu.SemaphoreType.DMA((NBUF,)),
        ],
    )(query_address, addresses)

    wsum = jnp.sum(outv, axis=0) + wsum_tc[0]
    ssum = jnp.sum(outs) + ssum_tc[0, 0]
    return wsum / ssum
